# Initial kernel scaffold; baseline (speedup 1.0000x reference)
import jax
import jax.numpy as jnp
from jax.experimental import pallas as pl


def kernel(so3_indices, scores, medians, mads):
    raise NotImplementedError("write your pallas kernel here")


# speedup vs baseline: 282.9563x; 282.9563x over previous
import functools

import jax
import jax.numpy as jnp
from jax import lax
from jax.experimental import pallas as pl
from jax.experimental.pallas import tpu as pltpu, tpu_sc as plsc

N_PSI = 192
N_CONES = 12288
N_PART = 1048576
NUM_CORES = 2
NUM_SUBCORES = 16
NW = NUM_CORES * NUM_SUBCORES
B_PER_W = N_PART // NW
L = 16

_mesh = plsc.VectorSubcoreMesh(core_axis_name="c", subcore_axis_name="s")


@functools.partial(
    pl.kernel,
    mesh=_mesh,
    out_type=jax.ShapeDtypeStruct((N_PART,), jnp.float32),
    scratch_types=[
        pltpu.VMEM((B_PER_W,), jnp.int32),
        pltpu.VMEM((B_PER_W,), jnp.float32),
        pltpu.VMEM((N_CONES,), jnp.float32),
        pltpu.VMEM((N_CONES,), jnp.float32),
    ],
    compiler_params=pltpu.CompilerParams(needs_layout_passes=False),
)
def _normalize(idx_hbm, scores_hbm, med_hbm, mad_hbm, out_hbm,
               idx_v, sc_v, med_v, mad_v):
    wid = lax.axis_index("s") * NUM_CORES + lax.axis_index("c")
    base = wid * B_PER_W
    pltpu.sync_copy(med_hbm, med_v)
    pltpu.sync_copy(mad_hbm, mad_v)
    pltpu.sync_copy(idx_hbm.at[pl.ds(base, B_PER_W)], idx_v)
    pltpu.sync_copy(scores_hbm.at[pl.ds(base, B_PER_W)], sc_v)

    def step(i, carry):
        off = i * L
        cone = idx_v[pl.ds(off, L)] // N_PSI
        med = plsc.load_gather(med_v, [cone])
        mad = plsc.load_gather(mad_v, [cone])
        sc_v[pl.ds(off, L)] = (sc_v[pl.ds(off, L)] - med) / mad
        return carry

    lax.fori_loop(0, B_PER_W // L, step, 0)
    pltpu.sync_copy(sc_v, out_hbm.at[pl.ds(base, B_PER_W)])


def kernel(so3_indices, scores, medians, mads):
    return _normalize(so3_indices, scores, medians, mads)


# speedup vs baseline: 452.5241x; 1.5993x over previous
import functools

import jax
import jax.numpy as jnp
from jax import lax
from jax.experimental import pallas as pl
from jax.experimental.pallas import tpu as pltpu, tpu_sc as plsc

N_PSI = 192
N_CONES = 12288
N_PART = 1048576
NUM_CORES = 2
NUM_SUBCORES = 16
NW = NUM_CORES * NUM_SUBCORES
B_PER_W = N_PART // NW
L = 16
UNROLL = 8

_mesh = plsc.VectorSubcoreMesh(core_axis_name="c", subcore_axis_name="s")


@functools.partial(
    pl.kernel,
    mesh=_mesh,
    out_type=jax.ShapeDtypeStruct((N_PART,), jnp.float32),
    scratch_types=[
        pltpu.VMEM((B_PER_W,), jnp.int32),
        pltpu.VMEM((B_PER_W,), jnp.float32),
        pltpu.VMEM((N_CONES,), jnp.float32),
        pltpu.VMEM((N_CONES,), jnp.float32),
    ],
    compiler_params=pltpu.CompilerParams(needs_layout_passes=False),
)
def _normalize(idx_hbm, scores_hbm, med_hbm, mad_hbm, out_hbm,
               idx_v, sc_v, med_v, mad_v):
    wid = lax.axis_index("s") * NUM_CORES + lax.axis_index("c")
    base = wid * B_PER_W
    pltpu.sync_copy(med_hbm, med_v)
    pltpu.sync_copy(mad_hbm, mad_v)
    pltpu.sync_copy(idx_hbm.at[pl.ds(base, B_PER_W)], idx_v)
    pltpu.sync_copy(scores_hbm.at[pl.ds(base, B_PER_W)], sc_v)

    def step(i, carry):
        base_off = i * (L * UNROLL)
        for u in range(UNROLL):
            off = base_off + u * L
            so3 = idx_v[pl.ds(off, L)]
            q6 = lax.shift_right_logical(so3, 6)
            cone = lax.shift_right_logical(q6 * jnp.int32(43691), 17)
            med = plsc.load_gather(med_v, [cone])
            mad = plsc.load_gather(mad_v, [cone])
            sc_v[pl.ds(off, L)] = (sc_v[pl.ds(off, L)] - med) / mad
        return carry

    lax.fori_loop(0, B_PER_W // (L * UNROLL), step, 0)
    pltpu.sync_copy(sc_v, out_hbm.at[pl.ds(base, B_PER_W)])


def kernel(so3_indices, scores, medians, mads):
    return _normalize(so3_indices, scores, medians, mads)


# speedup vs baseline: 528.9639x; 1.1689x over previous
import functools

import jax
import jax.numpy as jnp
from jax import lax
from jax.experimental import pallas as pl
from jax.experimental.pallas import tpu as pltpu, tpu_sc as plsc

N_PSI = 192
N_CONES = 12288
N_PART = 1048576
NUM_CORES = 2
NUM_SUBCORES = 16
NW = NUM_CORES * NUM_SUBCORES
B_PER_W = N_PART // NW
L = 16
UNROLL = 8

_mesh = plsc.VectorSubcoreMesh(core_axis_name="c", subcore_axis_name="s")


@functools.partial(
    pl.kernel,
    mesh=_mesh,
    out_type=jax.ShapeDtypeStruct((N_PART,), jnp.float32),
    scratch_types=[
        pltpu.VMEM((B_PER_W,), jnp.int32),
        pltpu.VMEM((B_PER_W,), jnp.float32),
        pltpu.VMEM((N_CONES,), jnp.float32),
        pltpu.VMEM((N_CONES,), jnp.float32),
    ],
    compiler_params=pltpu.CompilerParams(needs_layout_passes=False),
)
def _normalize(idx_hbm, scores_hbm, med_hbm, mad_hbm, out_hbm,
               idx_v, sc_v, med_v, mad_v):
    wid = lax.axis_index("s") * NUM_CORES + lax.axis_index("c")
    base = wid * B_PER_W
    pltpu.sync_copy(med_hbm, med_v)
    pltpu.sync_copy(mad_hbm, mad_v)
    pltpu.sync_copy(idx_hbm.at[pl.ds(base, B_PER_W)], idx_v)
    pltpu.sync_copy(scores_hbm.at[pl.ds(base, B_PER_W)], sc_v)

    def prep(j, carry):
        base_off = j * (L * UNROLL)
        for u in range(UNROLL):
            off = base_off + u * L
            r = 1.0 / mad_v[pl.ds(off, L)]
            mad_v[pl.ds(off, L)] = r
            med_v[pl.ds(off, L)] = med_v[pl.ds(off, L)] * r
        return carry

    lax.fori_loop(0, N_CONES // (L * UNROLL), prep, 0)

    def step(i, carry):
        base_off = i * (L * UNROLL)
        for u in range(UNROLL):
            off = base_off + u * L
            so3 = idx_v[pl.ds(off, L)]
            q6 = lax.shift_right_logical(so3, 6)
            cone = lax.shift_right_logical(q6 * jnp.int32(43691), 17)
            medr = plsc.load_gather(med_v, [cone])
            rmad = plsc.load_gather(mad_v, [cone])
            sc_v[pl.ds(off, L)] = sc_v[pl.ds(off, L)] * rmad - medr
        return carry

    lax.fori_loop(0, B_PER_W // (L * UNROLL), step, 0)
    pltpu.sync_copy(sc_v, out_hbm.at[pl.ds(base, B_PER_W)])


def kernel(so3_indices, scores, medians, mads):
    return _normalize(so3_indices, scores, medians, mads)


# speedup vs baseline: 697.6694x; 1.3189x over previous
import functools

import jax
import jax.numpy as jnp
from jax import lax
from jax.experimental import pallas as pl
from jax.experimental.pallas import tpu as pltpu, tpu_sc as plsc

N_PSI = 192
N_CONES = 12288
N_PART = 1048576
NUM_CORES = 2
NUM_SUBCORES = 16
NW = NUM_CORES * NUM_SUBCORES
B_PER_W = N_PART // NW
L = 16

_mesh = plsc.VectorSubcoreMesh(core_axis_name="c", subcore_axis_name="s")


@functools.partial(
    pl.kernel,
    mesh=_mesh,
    out_type=jax.ShapeDtypeStruct((N_PART,), jnp.float32),
    scratch_types=[
        pltpu.VMEM((B_PER_W,), jnp.int32),
        pltpu.VMEM((B_PER_W,), jnp.float32),
        pltpu.VMEM((N_CONES,), jnp.float32),
        pltpu.VMEM((N_CONES,), jnp.float32),
        pltpu.SemaphoreType.DMA,
        pltpu.SemaphoreType.DMA,
    ],
    compiler_params=pltpu.CompilerParams(needs_layout_passes=False),
)
def _normalize(idx_hbm, scores_hbm, med_hbm, mad_hbm, out_hbm,
               idx_v, sc_v, med_v, mad_v, sem_t, sem_d):
    wid = lax.axis_index("s") * NUM_CORES + lax.axis_index("c")
    base = wid * B_PER_W
    cp_med = pltpu.async_copy(med_hbm, med_v, sem_t)
    cp_mad = pltpu.async_copy(mad_hbm, mad_v, sem_t)
    cp_idx = pltpu.async_copy(idx_hbm.at[pl.ds(base, B_PER_W)], idx_v, sem_d)
    cp_sc = pltpu.async_copy(scores_hbm.at[pl.ds(base, B_PER_W)], sc_v, sem_d)
    cp_med.wait()
    cp_mad.wait()

    @plsc.parallel_loop(0, N_CONES, step=L, unroll=8)
    def _prep(off):
        r = 1.0 / mad_v[pl.ds(off, L)]
        mad_v[pl.ds(off, L)] = r
        med_v[pl.ds(off, L)] = med_v[pl.ds(off, L)] * r

    cp_idx.wait()
    cp_sc.wait()

    @plsc.parallel_loop(0, B_PER_W, step=L, unroll=8)
    def _step(off):
        so3 = idx_v[pl.ds(off, L)]
        q6 = lax.shift_right_logical(so3, 6)
        cone = lax.shift_right_logical(q6 * jnp.int32(43691), 17)
        medr = plsc.load_gather(med_v, [cone])
        rmad = plsc.load_gather(mad_v, [cone])
        sc_v[pl.ds(off, L)] = sc_v[pl.ds(off, L)] * rmad - medr

    pltpu.sync_copy(sc_v, out_hbm.at[pl.ds(base, B_PER_W)])


def kernel(so3_indices, scores, medians, mads):
    return _normalize(so3_indices, scores, medians, mads)


# speedup vs baseline: 707.6010x; 1.0142x over previous
import functools

import jax
import jax.numpy as jnp
from jax import lax
from jax.experimental import pallas as pl
from jax.experimental.pallas import tpu as pltpu, tpu_sc as plsc

N_PSI = 192
N_CONES = 12288
N_PART = 1048576
NUM_CORES = 2
NUM_SUBCORES = 16
NW = NUM_CORES * NUM_SUBCORES
B_PER_W = N_PART // NW
L = 16

_mesh = plsc.VectorSubcoreMesh(core_axis_name="c", subcore_axis_name="s")


def _rne_bf16_hi(u):
    odd = lax.shift_right_logical(u, 16) & jnp.int32(1)
    return u + jnp.int32(0x7FFF) + odd


@functools.partial(
    pl.kernel,
    mesh=_mesh,
    out_type=jax.ShapeDtypeStruct((N_PART,), jnp.float32),
    scratch_types=[
        pltpu.VMEM((B_PER_W,), jnp.int32),
        pltpu.VMEM((B_PER_W,), jnp.float32),
        pltpu.VMEM((N_CONES,), jnp.float32),
        pltpu.VMEM((N_CONES,), jnp.float32),
        pltpu.VMEM((N_CONES,), jnp.int32),
        pltpu.SemaphoreType.DMA,
        pltpu.SemaphoreType.DMA,
    ],
    compiler_params=pltpu.CompilerParams(needs_layout_passes=False),
)
def _normalize(idx_hbm, scores_hbm, med_hbm, mad_hbm, out_hbm,
               idx_v, sc_v, med_v, mad_v, pk_v, sem_t, sem_d):
    wid = lax.axis_index("s") * NUM_CORES + lax.axis_index("c")
    base = wid * B_PER_W
    cp_med = pltpu.async_copy(med_hbm, med_v, sem_t)
    cp_mad = pltpu.async_copy(mad_hbm, mad_v, sem_t)
    cp_idx = pltpu.async_copy(idx_hbm.at[pl.ds(base, B_PER_W)], idx_v, sem_d)
    cp_sc = pltpu.async_copy(scores_hbm.at[pl.ds(base, B_PER_W)], sc_v, sem_d)
    cp_med.wait()
    cp_mad.wait()

    @plsc.parallel_loop(0, N_CONES, step=L, unroll=8)
    def _prep(off):
        r = 1.0 / mad_v[pl.ds(off, L)]
        m = med_v[pl.ds(off, L)] * r
        rr = _rne_bf16_hi(plsc.bitcast(r, jnp.int32))
        rm = _rne_bf16_hi(plsc.bitcast(m, jnp.int32))
        pk_v[pl.ds(off, L)] = (rm & jnp.int32(-65536)) | lax.shift_right_logical(rr, 16)

    cp_idx.wait()
    cp_sc.wait()

    @plsc.parallel_loop(0, B_PER_W, step=L, unroll=8)
    def _step(off):
        so3 = idx_v[pl.ds(off, L)]
        q6 = lax.shift_right_logical(so3, 6)
        cone = lax.shift_right_logical(q6 * jnp.int32(43691), 17)
        w = plsc.load_gather(pk_v, [cone])
        rmad = plsc.bitcast(lax.shift_left(w, 16), jnp.float32)
        medr = plsc.bitcast(w & jnp.int32(-65536), jnp.float32)
        sc_v[pl.ds(off, L)] = sc_v[pl.ds(off, L)] * rmad - medr

    pltpu.sync_copy(sc_v, out_hbm.at[pl.ds(base, B_PER_W)])


def kernel(so3_indices, scores, medians, mads):
    return _normalize(so3_indices, scores, medians, mads)
